# baseline (speedup 1.0000x reference)
import functools

import jax
import jax.numpy as jnp
from jax import lax
from jax.experimental import pallas as pl
from jax.experimental.pallas import tpu as pltpu
from jax.experimental.pallas import tpu_sc as plsc

_ROWS = 32768
_COLS = 1024
_CW = 32
_CHUNK_STARTS = (0, 256, 512, 768)
_NC = 2
_NS = 16
_NW = _NC * _NS
_RPW = _ROWS // _NW

_mesh = plsc.VectorSubcoreMesh(
    core_axis_name="c", subcore_axis_name="s", num_cores=_NC,
    num_subcores=_NS)


@functools.partial(
    pl.kernel,
    out_type=[jax.ShapeDtypeStruct((_ROWS, _CW), jnp.float32)] * 4,
    mesh=_mesh,
    scratch_types=[pltpu.SemaphoreType.DMA],
    compiler_params=pltpu.CompilerParams(use_tc_tiling_on_sc=False),
)
def _chunk_select(x_hbm, o0, o1, o2, o3, sem):
    wid = lax.axis_index("s") * _NC + lax.axis_index("c")
    base = wid * _RPW
    outs = (o0, o1, o2, o3)
    copies = [
        pltpu.make_async_copy(
            x_hbm.at[pl.ds(base, _RPW), pl.ds(c, _CW)],
            outs[k].at[pl.ds(base, _RPW), :],
            sem,
        )
        for k, c in enumerate(_CHUNK_STARTS)
    ]
    for cp in copies:
        cp.start()
    for cp in copies:
        cp.wait()


def kernel(x):
    return tuple(_chunk_select(x))


# speedup vs baseline: 2.5787x; 2.5787x over previous
import functools

import jax
import jax.numpy as jnp
from jax import lax
from jax.experimental import pallas as pl
from jax.experimental.pallas import tpu as pltpu
from jax.experimental.pallas import tpu_sc as plsc

_ROWS = 32768
_COLS = 1024
_CW = 32
_CHUNK_STARTS = (0, 256, 512, 768)
_NC = 2
_NS = 16
_NW = _NC * _NS
_RPW = _ROWS // _NW
_BR = 64
_NB = _RPW // _BR

_mesh = plsc.VectorSubcoreMesh(
    core_axis_name="c", subcore_axis_name="s", num_cores=_NC,
    num_subcores=_NS)


@functools.partial(
    pl.kernel,
    out_type=[jax.ShapeDtypeStruct((_ROWS, _CW), jnp.float32)] * 4,
    mesh=_mesh,
    scratch_types=[
        pltpu.VMEM((_BR, _COLS), jnp.float32),
        pltpu.VMEM((_BR, _COLS), jnp.float32),
        pltpu.SemaphoreType.DMA,
        pltpu.SemaphoreType.DMA,
        pltpu.SemaphoreType.DMA,
        pltpu.SemaphoreType.DMA,
    ],
    compiler_params=pltpu.CompilerParams(use_tc_tiling_on_sc=False),
)
def _chunk_select(x_hbm, o0, o1, o2, o3, b0, b1, l0, l1, s0, s1):
    outs = (o0, o1, o2, o3)
    bufs = (b0, b1)
    lsems = (l0, l1)
    ssems = (s0, s1)
    wid = lax.axis_index("s") * _NC + lax.axis_index("c")
    base = wid * _RPW

    def load(i):
        return pltpu.make_async_copy(
            x_hbm.at[pl.ds(base + i * _BR, _BR), :], bufs[i % 2],
            lsems[i % 2])

    def stores(i):
        return [
            pltpu.make_async_copy(
                bufs[i % 2].at[:, pl.ds(c, _CW)],
                outs[k].at[pl.ds(base + i * _BR, _BR), :],
                ssems[i % 2])
            for k, c in enumerate(_CHUNK_STARTS)
        ]

    load(0).start()
    for i in range(_NB):
        load(i).wait()
        for s in stores(i):
            s.start()
        if i + 1 < _NB:
            if i >= 1:
                for s in stores(i - 1):
                    s.wait()
            load(i + 1).start()
    for i in (_NB - 2, _NB - 1):
        for s in stores(i):
            s.wait()


def kernel(x):
    return tuple(_chunk_select(x))


# speedup vs baseline: 3.3226x; 1.2885x over previous
import functools

import jax
import jax.numpy as jnp
import numpy as np
from jax import lax
from jax.experimental import pallas as pl
from jax.experimental.pallas import tpu as pltpu
from jax.experimental.pallas import tpu_sc as plsc

_ROWS = 32768
_COLS = 1024
_CW = 32
_CHUNK_STARTS = (0, 256, 512, 768)
_NCH = 4
_NC = 2
_NS = 16
_NW = _NC * _NS
_RPW = _ROWS // _NW

_mesh = plsc.VectorSubcoreMesh(
    core_axis_name="c", subcore_axis_name="s", num_cores=_NC,
    num_subcores=_NS)

_IDX = np.arange(_ROWS, dtype=np.int32) * (_COLS // _CW)


@functools.partial(
    pl.kernel,
    out_type=[jax.ShapeDtypeStruct((_ROWS, _CW), jnp.float32)] * _NCH,
    mesh=_mesh,
    scratch_types=[
        pltpu.VMEM((_RPW, _CW), jnp.float32),
        pltpu.VMEM((_RPW, _CW), jnp.float32),
        pltpu.VMEM((_RPW,), jnp.int32),
        pltpu.VMEM((_RPW,), jnp.int32),
        pltpu.SemaphoreType.DMA,
        pltpu.SemaphoreType.DMA,
        pltpu.SemaphoreType.DMA,
        pltpu.SemaphoreType.DMA,
    ],
    compiler_params=pltpu.CompilerParams(use_tc_tiling_on_sc=False),
)
def _chunk_select(xv_hbm, i0, i1, i2, i3, o0, o1, o2, o3,
                  b0, b1, v0, v1, g0, g1, s0, s1):
    outs = (o0, o1, o2, o3)
    idxs = (i0, i1, i2, i3)
    bufs = (b0, b1)
    ivs = (v0, v1)
    gsems = (g0, g1)
    ssems = (s0, s1)
    wid = lax.axis_index("s") * _NC + lax.axis_index("c")
    base = wid * _RPW

    def gather(k):
        pltpu.sync_copy(idxs[k].at[pl.ds(base, _RPW)], ivs[k % 2])
        pltpu.make_async_copy(
            xv_hbm.at[ivs[k % 2]], bufs[k % 2], gsems[k % 2]).start()

    def store(k):
        return pltpu.make_async_copy(
            bufs[k % 2], outs[k].at[pl.ds(base, _RPW), :], ssems[k % 2])

    gather(0)
    for k in range(_NCH):
        pltpu.make_async_copy(
            xv_hbm.at[ivs[k % 2]], bufs[k % 2], gsems[k % 2]).wait()
        store(k).start()
        if k + 1 < _NCH:
            if k >= 1:
                store(k - 1).wait()
            gather(k + 1)
    store(_NCH - 2).wait()
    store(_NCH - 1).wait()


def kernel(x):
    xv = x.reshape(_ROWS * (_COLS // _CW), _CW)
    idx = [jnp.asarray(_IDX + 8 * k) for k in range(_NCH)]
    return tuple(_chunk_select(xv, *idx))


# speedup vs baseline: 5.9932x; 1.8038x over previous
import functools

import jax
import jax.numpy as jnp
from jax import lax
from jax.experimental import pallas as pl
from jax.experimental.pallas import tpu as pltpu
from jax.experimental.pallas import tpu_sc as plsc

_ROWS = 32768
_COLS = 1024
_CW = 32
_BAND = 128
_CHUNK_STARTS = (0, 256, 512, 768)
_NCH = 4
_NC = 2
_NS = 16
_NW = _NC * _NS
_RPW = _ROWS // _NW
_BR = 256
_NBLK = _RPW // _BR
_L = 16

_mesh = plsc.VectorSubcoreMesh(
    core_axis_name="c", subcore_axis_name="s", num_cores=_NC,
    num_subcores=_NS)


@functools.partial(
    pl.kernel,
    out_type=[jax.ShapeDtypeStruct((_ROWS, _CW), jnp.float32)] * _NCH,
    mesh=_mesh,
    scratch_types=[
        pltpu.VMEM((_BR, _BAND), jnp.float32),
        pltpu.VMEM((_BR, _BAND), jnp.float32),
        pltpu.VMEM((_BR, _CW), jnp.float32),
        pltpu.VMEM((_BR, _CW), jnp.float32),
        pltpu.SemaphoreType.DMA,
        pltpu.SemaphoreType.DMA,
        pltpu.SemaphoreType.DMA,
        pltpu.SemaphoreType.DMA,
    ],
)
def _chunk_select(x_hbm, o0, o1, o2, o3,
                  b0, b1, c0, c1, l0, l1, s0, s1):
    outs = (o0, o1, o2, o3)
    bufs = (b0, b1)
    cbufs = (c0, c1)
    lsems = (l0, l1)
    ssems = (s0, s1)
    wid = lax.axis_index("s") * _NC + lax.axis_index("c")
    base = wid * _RPW
    niter = _NCH * _NBLK

    def rows(i):
        return pl.ds(base + (i % _NBLK) * _BR, _BR)

    def load(i):
        return pltpu.make_async_copy(
            x_hbm.at[rows(i), pl.ds(_CHUNK_STARTS[i // _NBLK], _BAND)],
            bufs[i % 2], lsems[i % 2])

    def store(i):
        return pltpu.make_async_copy(
            cbufs[i % 2], outs[i // _NBLK].at[rows(i), :], ssems[i % 2])

    def compact(i):
        src = bufs[i % 2]
        dst = cbufs[i % 2]

        def body(r, carry):
            dst[r, pl.ds(0, _L)] = src[r, pl.ds(0, _L)]
            dst[r, pl.ds(_L, _L)] = src[r, pl.ds(_L, _L)]
            return carry

        lax.fori_loop(0, _BR, body, 0, unroll=8)

    load(0).start()
    for i in range(niter):
        if i + 1 < niter:
            load(i + 1).start()
        load(i).wait()
        if i >= 2:
            store(i - 2).wait()
        compact(i)
        store(i).start()
    store(niter - 2).wait()
    store(niter - 1).wait()


def kernel(x):
    return tuple(_chunk_select(x))


# speedup vs baseline: 6.6336x; 1.1068x over previous
import functools

import jax
import jax.numpy as jnp
from jax import lax
from jax.experimental import pallas as pl
from jax.experimental.pallas import tpu as pltpu
from jax.experimental.pallas import tpu_sc as plsc

_ROWS = 32768
_COLS = 1024
_CW = 32
_BAND = 128
_CHUNK_STARTS = (0, 256, 512, 768)
_SC_CHUNKS = (0, 256)
_TC_CHUNKS = (512, 768)
_NC = 2
_NS = 16
_NW = _NC * _NS
_RPW = _ROWS // _NW
_BR = 256
_NBLK = _RPW // _BR
_L = 16

_mesh = plsc.VectorSubcoreMesh(
    core_axis_name="c", subcore_axis_name="s", num_cores=_NC,
    num_subcores=_NS)


@functools.partial(
    pl.kernel,
    out_type=[jax.ShapeDtypeStruct((_ROWS, _CW), jnp.float32)] * 2,
    mesh=_mesh,
    scratch_types=[
        pltpu.VMEM((_BR, _BAND), jnp.float32),
        pltpu.VMEM((_BR, _BAND), jnp.float32),
        pltpu.VMEM((_BR, _CW), jnp.float32),
        pltpu.VMEM((_BR, _CW), jnp.float32),
        pltpu.SemaphoreType.DMA,
        pltpu.SemaphoreType.DMA,
        pltpu.SemaphoreType.DMA,
        pltpu.SemaphoreType.DMA,
    ],
)
def _sc_select(x_hbm, o0, o1, b0, b1, c0, c1, l0, l1, s0, s1):
    outs = (o0, o1)
    bufs = (b0, b1)
    cbufs = (c0, c1)
    lsems = (l0, l1)
    ssems = (s0, s1)
    wid = lax.axis_index("s") * _NC + lax.axis_index("c")
    base = wid * _RPW
    niter = len(_SC_CHUNKS) * _NBLK

    def rows(i):
        return pl.ds(base + (i % _NBLK) * _BR, _BR)

    def load(i):
        return pltpu.make_async_copy(
            x_hbm.at[rows(i), pl.ds(_SC_CHUNKS[i // _NBLK], _BAND)],
            bufs[i % 2], lsems[i % 2])

    def store(i):
        return pltpu.make_async_copy(
            cbufs[i % 2], outs[i // _NBLK].at[rows(i), :], ssems[i % 2])

    def compact(i):
        src = bufs[i % 2]
        dst = cbufs[i % 2]

        def body(r, carry):
            dst[r, pl.ds(0, _L)] = src[r, pl.ds(0, _L)]
            dst[r, pl.ds(_L, _L)] = src[r, pl.ds(_L, _L)]
            return carry

        lax.fori_loop(0, _BR, body, 0, unroll=8)

    load(0).start()
    for i in range(niter):
        if i + 1 < niter:
            load(i + 1).start()
        load(i).wait()
        if i >= 2:
            store(i - 2).wait()
        compact(i)
        store(i).start()
    store(niter - 2).wait()
    store(niter - 1).wait()


_TC_R = 2048


def _tc_body(x2_ref, x3_ref, o2_ref, o3_ref):
    o2_ref[...] = x2_ref[:, 0:_CW]
    o3_ref[...] = x3_ref[:, 0:_CW]


_tc_select = pl.pallas_call(
    _tc_body,
    grid=(_ROWS // _TC_R,),
    in_specs=[
        pl.BlockSpec((_TC_R, _BAND), lambda i, c=c: (i, c // _BAND))
        for c in _TC_CHUNKS
    ],
    out_specs=[pl.BlockSpec((_TC_R, _CW), lambda i: (i, 0))] * 2,
    out_shape=[jax.ShapeDtypeStruct((_ROWS, _CW), jnp.float32)] * 2,
)


def kernel(x):
    o0, o1 = _sc_select(x)
    o2, o3 = _tc_select(x, x)
    return (o0, o1, o2, o3)


# speedup vs baseline: 6.6604x; 1.0040x over previous
import functools

import jax
import jax.numpy as jnp
from jax import lax
from jax.experimental import pallas as pl
from jax.experimental.pallas import tpu as pltpu
from jax.experimental.pallas import tpu_sc as plsc

_ROWS = 32768
_COLS = 1024
_CW = 32
_BAND = 128
_CHUNK_STARTS = (0, 256, 512, 768)
_SC_CHUNKS = (0, 256)
_TC_CHUNKS = (512, 768)
_NC = 2
_NS = 16
_NW = _NC * _NS
_RPW = _ROWS // _NW
_BR = 256
_NBLK = _RPW // _BR
_L = 16

_mesh = plsc.VectorSubcoreMesh(
    core_axis_name="c", subcore_axis_name="s", num_cores=_NC,
    num_subcores=_NS)


@functools.partial(
    pl.kernel,
    out_type=[jax.ShapeDtypeStruct((_ROWS, _CW), jnp.float32)] * 2,
    mesh=_mesh,
    scratch_types=[
        pltpu.VMEM((_BR, _BAND), jnp.float32),
        pltpu.VMEM((_BR, _BAND), jnp.float32),
        pltpu.VMEM((_BR, _CW), jnp.float32),
        pltpu.VMEM((_BR, _CW), jnp.float32),
        pltpu.SemaphoreType.DMA,
        pltpu.SemaphoreType.DMA,
        pltpu.SemaphoreType.DMA,
        pltpu.SemaphoreType.DMA,
    ],
)
def _sc_select(x_hbm, o0, o1, b0, b1, c0, c1, l0, l1, s0, s1):
    outs = (o0, o1)
    bufs = (b0, b1)
    cbufs = (c0, c1)
    lsems = (l0, l1)
    ssems = (s0, s1)
    wid = lax.axis_index("s") * _NC + lax.axis_index("c")
    base = wid * _RPW
    niter = len(_SC_CHUNKS) * _NBLK

    def rows(i):
        return pl.ds(base + (i % _NBLK) * _BR, _BR)

    def load(i):
        return pltpu.make_async_copy(
            x_hbm.at[rows(i), pl.ds(_SC_CHUNKS[i // _NBLK], _BAND)],
            bufs[i % 2], lsems[i % 2])

    def store(i):
        return pltpu.make_async_copy(
            cbufs[i % 2], outs[i // _NBLK].at[rows(i), :], ssems[i % 2])

    def compact(i):
        src = bufs[i % 2]
        dst = cbufs[i % 2]

        def body(r, carry):
            dst[r, pl.ds(0, _L)] = src[r, pl.ds(0, _L)]
            dst[r, pl.ds(_L, _L)] = src[r, pl.ds(_L, _L)]
            return carry

        lax.fori_loop(0, _BR, body, 0, unroll=8)

    load(0).start()
    for i in range(niter):
        if i + 1 < niter:
            load(i + 1).start()
        load(i).wait()
        if i >= 2:
            store(i - 2).wait()
        compact(i)
        store(i).start()
    store(niter - 2).wait()
    store(niter - 1).wait()


_TC_R = 2048


def _tc_body(x2_ref, x3_ref, o2_ref, o3_ref):
    o2_ref[...] = x2_ref[...]
    o3_ref[...] = x3_ref[...]


_tc_select = pl.pallas_call(
    _tc_body,
    grid=(_ROWS // _TC_R,),
    in_specs=[
        pl.BlockSpec((_TC_R, _BAND), lambda i, c=c: (i, c // _BAND))
        for c in _TC_CHUNKS
    ],
    out_specs=[pl.BlockSpec((_TC_R, _BAND), lambda i: (i, 0))] * 2,
    out_shape=[jax.ShapeDtypeStruct((_ROWS, _BAND), jnp.float32)] * 2,
)


def kernel(x):
    o0, o1 = _sc_select(x)
    p2, p3 = _tc_select(x, x)
    return (o0, o1, p2[:, :_CW], p3[:, :_CW])
